# baseline (speedup 1.0000x reference)
import functools

import jax
import jax.numpy as jnp
from jax import lax
from jax.experimental import pallas as pl
from jax.experimental.pallas import tpu as pltpu
from jax.experimental.pallas import tpu_sc as plsc

_K = 1024
_D = 64
_T = 1024
_N = 8
_DC = 16
_NW = 32
_L = 16


def _argmin_body(z_ref, w_ref, q_ref):
    z = z_ref[0]
    w = w_ref[...]
    s = lax.dot_general(w, z, (((1,), (0,)), ((), ())),
                        preferred_element_type=jnp.float32)
    z2 = jnp.sum(z * z, axis=0)
    w2 = jnp.sum(w * w, axis=1)
    dist = (z2[None, :] - 2.0 * s) + w2[:, None]
    mind = jnp.min(dist, axis=0)
    kiota = lax.broadcasted_iota(jnp.int32, (_K, _T), 0)
    q_ref[0, 0] = jnp.min(jnp.where(dist == mind[None, :], kiota, _K), axis=0)


def _tc_argmin(zc, weights):
    return pl.pallas_call(
        _argmin_body,
        grid=(_N,),
        in_specs=[
            pl.BlockSpec((1, _D, _T), lambda n: (n, 0, 0)),
            pl.BlockSpec((_K, _D), lambda n: (0, 0)),
        ],
        out_specs=pl.BlockSpec((1, 1, _T), lambda n: (n, 0, 0)),
        out_shape=jax.ShapeDtypeStruct((_N, 1, _T), jnp.int32),
    )(zc, weights)


def _sc_gather_body(wt_hbm, q_hbm, out_hbm, idx_v, wt_v, out_v):
    cid = lax.axis_index("c")
    sid = lax.axis_index("s")
    wid = sid * 2 + cid
    n = wid // (_D // _DC)
    d0 = (wid % (_D // _DC)) * _DC
    pltpu.sync_copy(q_hbm.at[pl.ds(n * _T, _T)], idx_v)
    pltpu.sync_copy(wt_hbm.at[pl.ds(d0 * _K, _DC * _K)], wt_v)

    def j_body(j, carry):
        qv = idx_v[pl.ds(j * _L, _L)]
        for d in range(_DC):
            vals = plsc.load_gather(wt_v, [qv + jnp.int32(d * _K)])
            out_v[pl.ds(d * _T + j * _L, _L)] = vals
        return carry

    lax.fori_loop(0, _T // _L, j_body, 0)
    pltpu.sync_copy(out_v, out_hbm.at[n, pl.ds(d0 * _T, _DC * _T)])


@functools.partial(
    pl.kernel,
    mesh=plsc.VectorSubcoreMesh(core_axis_name="c", subcore_axis_name="s"),
    compiler_params=pltpu.CompilerParams(needs_layout_passes=False),
    out_type=jax.ShapeDtypeStruct((_N, _D * _T), jnp.float32),
    scratch_types=[
        pltpu.VMEM((_T,), jnp.int32),
        pltpu.VMEM((_DC * _K,), jnp.float32),
        pltpu.VMEM((_DC * _T,), jnp.float32),
    ],
)
def _sc_gather(wt_hbm, q_hbm, out_hbm, idx_v, wt_v, out_v):
    _sc_gather_body(wt_hbm, q_hbm, out_hbm, idx_v, wt_v, out_v)


def kernel(z_e, weights):
    N, D, H, W = z_e.shape
    T = H * W
    zc = z_e.reshape(N, D, T)
    q3 = _tc_argmin(zc, weights)
    zq = _sc_gather(weights.T.reshape(D * _K), q3.reshape(N * T))
    return q3.reshape(N, H, W), zq.reshape(N, D, H, W)


# speedup vs baseline: 2.4155x; 2.4155x over previous
import jax
import jax.numpy as jnp
from jax import lax
from jax.experimental import pallas as pl
from jax.experimental.pallas import tpu as pltpu

_K = 1024
_D = 64
_T = 1024
_R = 8


def _vq_body(z_ref, w_ref, w2x_ref, wt_ref, q_ref, zq_ref):
    z = z_ref[0]
    w = w_ref[...]
    w2x = w2x_ref[...]
    wt = wt_ref[...]
    s2 = lax.dot_general(w2x, z, (((1,), (0,)), ((), ())),
                         preferred_element_type=jnp.float32)
    z2 = jnp.sum(z * z, axis=0)
    w2 = jnp.sum(w * w, axis=1)
    z2b = z2[None, :]
    riota = lax.broadcasted_iota(jnp.int32, (_R, _T), 0)

    mv = jnp.full((_R, _T), jnp.inf, jnp.float32)
    mi = jnp.zeros((_R, _T), jnp.int32)
    for kc in range(_K // _R):
        k0 = kc * _R
        sc = lax.slice(s2, (k0, 0), (k0 + _R, _T))
        w2c = lax.slice(w2, (k0,), (k0 + _R,))
        d = (z2b - sc) + w2c[:, None]
        better = d < mv
        mv = jnp.where(better, d, mv)
        mi = jnp.where(better, riota + k0, mi)
    fmin = jnp.min(mv, axis=0)
    q = jnp.min(jnp.where(mv == fmin[None, :], mi, _K), axis=0)
    q_ref[0, 0] = q
    kiota = lax.broadcasted_iota(jnp.int32, (_K, _T), 0)
    ohf = (kiota == q[None, :]).astype(jnp.float32)
    zq = lax.dot_general(wt, ohf, (((1,), (0,)), ((), ())),
                         preferred_element_type=jnp.float32)
    zq_ref[0] = zq


def kernel(z_e, weights):
    N, D, H, W = z_e.shape
    T = H * W
    zc = z_e.reshape(N, D, T)
    q3, zq = pl.pallas_call(
        _vq_body,
        grid=(N,),
        in_specs=[
            pl.BlockSpec((1, D, T), lambda n: (n, 0, 0)),
            pl.BlockSpec((_K, D), lambda n: (0, 0)),
            pl.BlockSpec((_K, D), lambda n: (0, 0)),
            pl.BlockSpec((D, _K), lambda n: (0, 0)),
        ],
        out_specs=[
            pl.BlockSpec((1, 1, T), lambda n: (n, 0, 0)),
            pl.BlockSpec((1, D, T), lambda n: (n, 0, 0)),
        ],
        out_shape=[
            jax.ShapeDtypeStruct((N, 1, T), jnp.int32),
            jax.ShapeDtypeStruct((N, D, T), jnp.float32),
        ],
    )(zc, weights, weights * 2.0, weights.T)
    return q3.reshape(N, H, W), zq.reshape(N, D, H, W)
